# baseline (speedup 1.0000x reference)
import functools

import jax
import jax.numpy as jnp
from jax import lax
from jax.experimental import pallas as pl
from jax.experimental.pallas import tpu as pltpu
from jax.experimental.pallas import tpu_sc as plsc

BUCKET_NUMBER = 1000
MIN_VALUE = 0.0
MAX_VALUE = 1.0
STEP = (MAX_VALUE - MIN_VALUE) / BUCKET_NUMBER
EMBED_DIM = 64

LANES = 16
NB = 8
NBUF = 2


def _make_kernel(BATCH, H, D):
    info = plsc.get_sparse_core_info()
    NC, NS = info.num_cores, info.num_subcores
    NW = NC * NS
    assert BATCH % (NW * NB * NBUF) == 0
    rows_per_w = BATCH // NW
    n_iters = rows_per_w // (NB * NBUF)
    HPAD = 64

    mesh = plsc.VectorSubcoreMesh(core_axis_name="c", subcore_axis_name="s")

    @functools.partial(
        pl.kernel,
        out_type=jax.ShapeDtypeStruct((BATCH, H, D), jnp.float32),
        mesh=mesh,
        scratch_types=[
            pltpu.VMEM((NBUF, NB * H + LANES), jnp.float32),
            pltpu.VMEM((NBUF, NB, HPAD), jnp.int32),
            pltpu.VMEM((NBUF, NB, 56, D), jnp.float32),
            pltpu.SemaphoreType.DMA,
        ]
        + [pltpu.SemaphoreType.DMA for _ in range(NBUF)],
        compiler_params=pltpu.CompilerParams(use_tc_tiling_on_sc=False),
    )
    def k(x_hbm, table_hbm, out_hbm, x_v, idx_v, rows_v, gsem, *ssems):
        wid = lax.axis_index("s") * NC + lax.axis_index("c")
        b0 = wid * rows_per_w

        def iter_body(t, carry):
            handles = []
            for kb in range(NBUF):
                bstart = b0 + (t * NBUF + kb) * NB
                pltpu.sync_copy(x_hbm.at[pl.ds(bstart * H, NB * H)],
                                x_v.at[kb, pl.ds(0, NB * H)])
                for m in range(NB):
                    for i in range(4):
                        v = x_v[kb, pl.ds(m * H + i * LANES, LANES)]
                        t_ = (v - MIN_VALUE) / STEP
                        idx = t_.astype(jnp.int32)
                        idx = jnp.minimum(jnp.maximum(idx, 0),
                                          BUCKET_NUMBER - 1)
                        idx_v[kb, m, pl.ds(i * LANES, LANES)] = idx

                @pl.when(t > 0)
                def _(kb=kb, bstart=bstart):
                    pltpu.make_async_copy(
                        rows_v.at[kb, :, pl.ds(0, H)],
                        out_hbm.at[pl.ds(bstart - NBUF * NB, NB)],
                        ssems[kb],
                    ).wait()

                handles.append([
                    pltpu.async_copy(
                        table_hbm.at[idx_v.at[kb, m, pl.ds(0, 56)]],
                        rows_v.at[kb, m],
                        gsem,
                    )
                    for m in range(NB)
                ])
            for kb in range(NBUF):
                bstart = b0 + (t * NBUF + kb) * NB
                for h in handles[kb]:
                    h.wait()
                pltpu.async_copy(
                    rows_v.at[kb, :, pl.ds(0, H)],
                    out_hbm.at[pl.ds(bstart, NB)],
                    ssems[kb],
                )
            return carry

        lax.fori_loop(0, n_iters, iter_body, 0)

        for kb in range(NBUF):
            bstart = b0 + ((n_iters - 1) * NBUF + kb) * NB
            pltpu.make_async_copy(
                rows_v.at[kb, :, pl.ds(0, H)],
                out_hbm.at[pl.ds(bstart, NB)],
                ssems[kb],
            ).wait()

    return k


def kernel(x, table):
    if x.ndim == 2 and x.shape[1] == 1:
        x = jnp.squeeze(x, axis=-1)
    BATCH, H = x.shape
    D = table.shape[1]
    xf = x.reshape(BATCH * H)
    return _make_kernel(BATCH, H, D)(xf, table)
